# baseline (speedup 1.0000x reference)
import jax
import jax.numpy as jnp
from jax.experimental import pallas as pl
from jax.experimental.pallas import tpu as pltpu

QUEUE_ROWS = 32768
CHUNK = 2048
NBUF = 4


def _fifo_copy(x_ref, q_ref, o_ref, buf, sin, sout):
    shift = 4096
    keep = QUEUE_ROWS - shift
    n_q = keep // CHUNK
    n_chunks = QUEUE_ROWS // CHUNK

    ins = []
    outs = []
    for c in range(n_chunks):
        b = c % NBUF
        if c < n_q:
            src = q_ref.at[pl.ds(shift + c * CHUNK, CHUNK)]
        else:
            src = x_ref.at[pl.ds((c - n_q) * CHUNK, CHUNK)]
        ins.append(pltpu.make_async_copy(src, buf.at[b], sin.at[b]))
        outs.append(pltpu.make_async_copy(
            buf.at[b], o_ref.at[pl.ds(c * CHUNK, CHUNK)], sout.at[b]))

    for c in range(NBUF):
        ins[c].start()
    for c in range(n_chunks):
        ins[c].wait()
        outs[c].start()
        nxt = c + NBUF
        if nxt < n_chunks:
            outs[c].wait()
            ins[nxt].start()
    for c in range(n_chunks - NBUF, n_chunks):
        outs[c].wait()


def kernel(x, queue):
    return pl.pallas_call(
        _fifo_copy,
        out_shape=jax.ShapeDtypeStruct(queue.shape, queue.dtype),
        in_specs=[
            pl.BlockSpec(memory_space=pl.ANY),
            pl.BlockSpec(memory_space=pl.ANY),
        ],
        out_specs=pl.BlockSpec(memory_space=pl.ANY),
        scratch_shapes=[
            pltpu.VMEM((NBUF, CHUNK, 128), jnp.float32),
            pltpu.SemaphoreType.DMA((NBUF,)),
            pltpu.SemaphoreType.DMA((NBUF,)),
        ],
    )(x, queue)


# speedup vs baseline: 1.2621x; 1.2621x over previous
import jax
import jax.numpy as jnp
from jax.experimental import pallas as pl
from jax.experimental.pallas import tpu as pltpu

QUEUE_ROWS = 32768
CHUNK = 1024
NBUF = 8
LAG = 4


def _fifo_copy(x_ref, q_ref, o_ref, buf, sin, sout):
    shift = 4096
    keep = QUEUE_ROWS - shift
    n_q = keep // CHUNK
    n_chunks = QUEUE_ROWS // CHUNK

    ins = []
    outs = []
    for c in range(n_chunks):
        b = c % NBUF
        if c < n_q:
            src = q_ref.at[pl.ds(shift + c * CHUNK, CHUNK)]
        else:
            src = x_ref.at[pl.ds((c - n_q) * CHUNK, CHUNK)]
        ins.append(pltpu.make_async_copy(src, buf.at[b], sin.at[b]))
        outs.append(pltpu.make_async_copy(
            buf.at[b], o_ref.at[pl.ds(c * CHUNK, CHUNK)], sout.at[b]))

    started = NBUF
    for c in range(NBUF):
        ins[c].start()
    for c in range(n_chunks):
        ins[c].wait()
        outs[c].start()
        prev = c - LAG
        if prev >= 0:
            outs[prev].wait()
            if started < n_chunks:
                ins[started].start()
                started += 1
    for c in range(n_chunks - LAG, n_chunks):
        outs[c].wait()


def kernel(x, queue):
    return pl.pallas_call(
        _fifo_copy,
        out_shape=jax.ShapeDtypeStruct(queue.shape, queue.dtype),
        in_specs=[
            pl.BlockSpec(memory_space=pl.ANY),
            pl.BlockSpec(memory_space=pl.ANY),
        ],
        out_specs=pl.BlockSpec(memory_space=pl.ANY),
        scratch_shapes=[
            pltpu.VMEM((NBUF, CHUNK, 128), jnp.float32),
            pltpu.SemaphoreType.DMA((NBUF,)),
            pltpu.SemaphoreType.DMA((NBUF,)),
        ],
    )(x, queue)


# speedup vs baseline: 1.2953x; 1.0263x over previous
import functools

import jax
import jax.numpy as jnp
from jax.experimental import pallas as pl
from jax.experimental.pallas import tpu as pltpu

QUEUE_ROWS = 32768
BLOCK = 4096


def _fifo_copy(q_ref, x_ref, o_ref, *, n_q_blocks):
    i = pl.program_id(0)

    @pl.when(i < n_q_blocks)
    def _():
        o_ref[...] = q_ref[...]

    @pl.when(i >= n_q_blocks)
    def _():
        o_ref[...] = x_ref[...]


def kernel(x, queue):
    shift = x.shape[0]
    assert shift % BLOCK == 0 and QUEUE_ROWS % BLOCK == 0
    n_blocks = QUEUE_ROWS // BLOCK
    n_x_blocks = shift // BLOCK
    n_q_blocks = n_blocks - n_x_blocks
    shift_blocks = shift // BLOCK

    return pl.pallas_call(
        functools.partial(_fifo_copy, n_q_blocks=n_q_blocks),
        grid=(n_blocks,),
        in_specs=[
            pl.BlockSpec(
                (BLOCK, queue.shape[1]),
                lambda i: (jnp.minimum(i + shift_blocks, n_blocks - 1), 0),
            ),
            pl.BlockSpec(
                (BLOCK, x.shape[1]),
                lambda i: (jnp.clip(i - n_q_blocks, 0, n_x_blocks - 1), 0),
            ),
        ],
        out_specs=pl.BlockSpec((BLOCK, queue.shape[1]), lambda i: (i, 0)),
        out_shape=jax.ShapeDtypeStruct(queue.shape, queue.dtype),
        compiler_params=pltpu.CompilerParams(
            dimension_semantics=("arbitrary",),
        ),
    )(queue, x)


# speedup vs baseline: 1.4435x; 1.1144x over previous
import jax
import jax.numpy as jnp
from jax.experimental import pallas as pl
from jax.experimental.pallas import tpu as pltpu

QUEUE_ROWS = 32768
CHUNK = 4096
NBUF = 4
LAG = 2


def _fifo_copy(x_ref, q_ref, o_ref, buf, sin, sout):
    shift = 4096
    keep = QUEUE_ROWS - shift
    n_q = keep // CHUNK
    n_chunks = QUEUE_ROWS // CHUNK

    ins = []
    outs = []
    for c in range(n_chunks):
        b = c % NBUF
        if c < n_q:
            src = q_ref.at[pl.ds(shift + c * CHUNK, CHUNK)]
        else:
            src = x_ref.at[pl.ds((c - n_q) * CHUNK, CHUNK)]
        ins.append(pltpu.make_async_copy(src, buf.at[b], sin.at[b]))
        outs.append(pltpu.make_async_copy(
            buf.at[b], o_ref.at[pl.ds(c * CHUNK, CHUNK)], sout.at[b]))

    started = NBUF
    for c in range(NBUF):
        ins[c].start()
    for c in range(n_chunks):
        ins[c].wait()
        outs[c].start()
        prev = c - LAG
        if prev >= 0:
            outs[prev].wait()
            if started < n_chunks:
                ins[started].start()
                started += 1
    for c in range(n_chunks - LAG, n_chunks):
        outs[c].wait()


def kernel(x, queue):
    return pl.pallas_call(
        _fifo_copy,
        out_shape=jax.ShapeDtypeStruct(queue.shape, queue.dtype),
        in_specs=[
            pl.BlockSpec(memory_space=pl.ANY),
            pl.BlockSpec(memory_space=pl.ANY),
        ],
        out_specs=pl.BlockSpec(memory_space=pl.ANY),
        scratch_shapes=[
            pltpu.VMEM((NBUF, CHUNK, 128), jnp.float32),
            pltpu.SemaphoreType.DMA((NBUF,)),
            pltpu.SemaphoreType.DMA((NBUF,)),
        ],
    )(x, queue)


# speedup vs baseline: 1.5880x; 1.1002x over previous
import jax
import jax.numpy as jnp
from jax.experimental import pallas as pl
from jax.experimental.pallas import tpu as pltpu

QUEUE_ROWS = 32768
CHUNK = 4096
NBUF = 8


def _fifo_copy(x_ref, q_ref, o_ref, buf, sin, sout):
    shift = 4096
    keep = QUEUE_ROWS - shift
    n_q = keep // CHUNK
    n_chunks = QUEUE_ROWS // CHUNK

    ins = []
    outs = []
    for c in range(n_chunks):
        b = c % NBUF
        if c < n_q:
            src = q_ref.at[pl.ds(shift + c * CHUNK, CHUNK)]
        else:
            src = x_ref.at[pl.ds((c - n_q) * CHUNK, CHUNK)]
        ins.append(pltpu.make_async_copy(src, buf.at[b], sin.at[b]))
        outs.append(pltpu.make_async_copy(
            buf.at[b], o_ref.at[pl.ds(c * CHUNK, CHUNK)], sout.at[b]))

    for c in range(n_chunks):
        ins[c].start()
    for c in range(n_chunks):
        ins[c].wait()
        outs[c].start()
    for c in range(n_chunks):
        outs[c].wait()


def kernel(x, queue):
    return pl.pallas_call(
        _fifo_copy,
        out_shape=jax.ShapeDtypeStruct(queue.shape, queue.dtype),
        in_specs=[
            pl.BlockSpec(memory_space=pl.ANY),
            pl.BlockSpec(memory_space=pl.ANY),
        ],
        out_specs=pl.BlockSpec(memory_space=pl.ANY),
        scratch_shapes=[
            pltpu.VMEM((NBUF, CHUNK, 128), jnp.float32),
            pltpu.SemaphoreType.DMA((NBUF,)),
            pltpu.SemaphoreType.DMA((NBUF,)),
        ],
    )(x, queue)
